# baseline (speedup 1.0000x reference)
import jax
import jax.numpy as jnp
from jax.experimental import pallas as pl


_TB = 512


def _add_kernel(tok_ref, pos_ref, out_ref):
    out_ref[...] = tok_ref[...] + pos_ref[...]


def kernel(tokens, pos_table):
    B, T, D = tokens.shape
    grid = (T // _TB, B)
    return pl.pallas_call(
        _add_kernel,
        grid=grid,
        in_specs=[
            pl.BlockSpec((1, _TB, D), lambda t, b: (b, t, 0)),
            pl.BlockSpec((_TB, D), lambda t, b: (t, 0)),
        ],
        out_specs=pl.BlockSpec((1, _TB, D), lambda t, b: (b, t, 0)),
        out_shape=jax.ShapeDtypeStruct((B, T, D), tokens.dtype),
    )(tokens, pos_table)


# speedup vs baseline: 1.2430x; 1.2430x over previous
import jax
import jax.numpy as jnp
from jax.experimental import pallas as pl


_TB = 2048


def _add_kernel(tok_ref, pos_ref, out_ref):
    out_ref[...] = tok_ref[...] + pos_ref[...]


def kernel(tokens, pos_table):
    B, T, D = tokens.shape
    grid = (T // _TB, B)
    return pl.pallas_call(
        _add_kernel,
        grid=grid,
        in_specs=[
            pl.BlockSpec((1, _TB, D), lambda t, b: (b, t, 0)),
            pl.BlockSpec((_TB, D), lambda t, b: (t, 0)),
        ],
        out_specs=pl.BlockSpec((1, _TB, D), lambda t, b: (b, t, 0)),
        out_shape=jax.ShapeDtypeStruct((B, T, D), tokens.dtype),
    )(tokens, pos_table)
